# baseline (speedup 1.0000x reference)
import functools

import jax
import jax.numpy as jnp
from jax import lax
from jax.experimental import pallas as pl
from jax.experimental.pallas import tpu as pltpu
from jax.experimental.pallas import tpu_sc as plsc

N = 10000
K = 32
D = 128
E = N * K
L = 16
C = D // L
NB = 8
IR = NB * K // 128
NUM_BATCHES = N // NB
NW = 32
BATCHES_PER_W = -(-NUM_BATCHES // NW)
BR = 1000


def _tc_linear(feature, W_u, b_u, W_v, b_v):
    def body(x_ref, wu_ref, bu_ref, wv_ref, bv_ref, u_ref, v_ref):
        x = x_ref[...]
        u_ref[...] = (lax.dot_general(
            x, wu_ref[...], (((1,), (1,)), ((), ())),
            preferred_element_type=jnp.float32) + bu_ref[...]) * (1.0 / K)
        v_ref[...] = (lax.dot_general(
            x, wv_ref[...], (((1,), (1,)), ((), ())),
            preferred_element_type=jnp.float32) + bv_ref[...]) * (1.0 / K)

    return pl.pallas_call(
        body,
        grid=(N // BR,),
        in_specs=[
            pl.BlockSpec((BR, D), lambda i: (i, 0)),
            pl.BlockSpec((D, D), lambda i: (0, 0)),
            pl.BlockSpec((1, D), lambda i: (0, 0)),
            pl.BlockSpec((D, D), lambda i: (0, 0)),
            pl.BlockSpec((1, D), lambda i: (0, 0)),
        ],
        out_specs=[
            pl.BlockSpec((BR, D), lambda i: (i, 0)),
            pl.BlockSpec((BR, D), lambda i: (i, 0)),
        ],
        out_shape=[
            jax.ShapeDtypeStruct((N, D), jnp.float32),
            jax.ShapeDtypeStruct((N, D), jnp.float32),
        ],
    )(feature, W_u, b_u.reshape(1, D), W_v, b_v.reshape(1, D))


def _sc_edges(u_feat, v_feat, src, sp, wvec, bvec):
    mesh = plsc.VectorSubcoreMesh(core_axis_name="c", subcore_axis_name="s")
    TRIPS = (BATCHES_PER_W + 2) // 3

    @functools.partial(
        pl.kernel,
        out_type=[
            jax.ShapeDtypeStruct((N, K), jnp.float32),
            jax.ShapeDtypeStruct((N, D), jnp.float32),
        ],
        mesh=mesh,
        compiler_params=pltpu.CompilerParams(needs_layout_passes=False),
        scratch_types=[
            pltpu.VMEM((IR, 128), jnp.int32),
            pltpu.VMEM((IR, 128), jnp.int32),
            pltpu.VMEM((IR, 128), jnp.int32),
            pltpu.VMEM((NB * K,), jnp.float32),
            pltpu.VMEM((NB * K,), jnp.float32),
            pltpu.VMEM((NB * K,), jnp.float32),
            pltpu.VMEM((NB, D), jnp.float32),
            pltpu.VMEM((NB, D), jnp.float32),
            pltpu.VMEM((NB, D), jnp.float32),
            pltpu.VMEM((NB * K + 1, D), jnp.float32),
            pltpu.VMEM((NB * K + 1, D), jnp.float32),
            pltpu.VMEM((NB * K + 1, D), jnp.float32),
            pltpu.VMEM((L, NB * K), jnp.float32),
            pltpu.VMEM((D,), jnp.float32),
            pltpu.VMEM((L,), jnp.float32),
            pltpu.VMEM((NB, K), jnp.float32),
            pltpu.VMEM((NB, K), jnp.float32),
            pltpu.VMEM((NB, K), jnp.float32),
            pltpu.VMEM((NB, D), jnp.float32),
            pltpu.VMEM((NB, D), jnp.float32),
            pltpu.VMEM((NB, D), jnp.float32),
            pltpu.SemaphoreType.DMA,
            pltpu.SemaphoreType.DMA,
            pltpu.SemaphoreType.DMA,
            pltpu.SemaphoreType.DMA,
            pltpu.SemaphoreType.DMA,
            pltpu.SemaphoreType.DMA,
            pltpu.SemaphoreType.DMA,
            pltpu.SemaphoreType.DMA,
            pltpu.SemaphoreType.DMA,
        ],
    )
    def k(u_hbm, v_hbm, src_hbm, sp_hbm, w_hbm, b_hbm, pos_hbm, str_hbm,
          idx_v0, idx_v1, idx_v2, sp_v0, sp_v1, sp_v2, vr0, vr1, vr2,
          rows0, rows1, rows2, parts, w_v, b_v,
          pos_s0, pos_s1, pos_s2, str_s0, str_s1, str_s2,
          sem_a0, sem_a1, sem_a2, sem_g0, sem_g1, sem_g2,
          sem_o0, sem_o1, sem_o2):
        idx_v = (idx_v0, idx_v1, idx_v2)
        sp_v = (sp_v0, sp_v1, sp_v2)
        vrows = (vr0, vr1, vr2)
        rows = (rows0, rows1, rows2)
        pos_s = (pos_s0, pos_s1, pos_s2)
        str_s = (str_s0, str_s1, str_s2)
        sem_a = (sem_a0, sem_a1, sem_a2)
        sem_g = (sem_g0, sem_g1, sem_g2)
        sem_o = (sem_o0, sem_o1, sem_o2)

        wid = lax.axis_index("s") * 2 + lax.axis_index("c")
        pltpu.sync_copy(w_hbm, w_v)
        pltpu.sync_copy(b_hbm, b_v)
        iota = lax.iota(jnp.int32, L)

        def start_aux(b, p):
            nb = b * NB
            pltpu.async_copy(src_hbm.at[pl.ds(b * IR, IR)], idx_v[p], sem_a[p])
            pltpu.async_copy(sp_hbm.at[pl.ds(nb * K, NB * K)], sp_v[p], sem_a[p])
            pltpu.async_copy(v_hbm.at[pl.ds(nb, NB)], vrows[p], sem_a[p])

        def wait_aux(p):
            pltpu.make_async_copy(src_hbm.at[pl.ds(0, IR)], idx_v[p], sem_a[p]).wait()
            pltpu.make_async_copy(sp_hbm.at[pl.ds(0, NB * K)], sp_v[p], sem_a[p]).wait()
            pltpu.make_async_copy(v_hbm.at[pl.ds(0, NB)], vrows[p], sem_a[p]).wait()

        def start_gather(p):
            for h in range(IR):
                pltpu.async_copy(u_hbm.at[idx_v[p].at[h]],
                                 rows[p].at[pl.ds(h * 128, 128)], sem_g[p])

        def wait_gather(p):
            for h in range(IR):
                pltpu.make_async_copy(u_hbm.at[idx_v[p].at[h]],
                                      rows[p].at[pl.ds(h * 128, 128)],
                                      sem_g[p]).wait()

        def start_out(b, p):
            nb = b * NB
            pltpu.async_copy(pos_s[p], pos_hbm.at[pl.ds(nb, NB)], sem_o[p])
            pltpu.async_copy(str_s[p], str_hbm.at[pl.ds(nb, NB)], sem_o[p])

        def wait_out(p):
            pltpu.make_async_copy(pos_s[p], pos_hbm.at[pl.ds(0, NB)], sem_o[p]).wait()
            pltpu.make_async_copy(str_s[p], str_hbm.at[pl.ds(0, NB)], sem_o[p]).wait()

        gdn = lax.GatherDimensionNumbers(
            offset_dims=(), collapsed_slice_dims=(0,), start_index_map=(0,))

        def compute(p):
            wch = [w_v[pl.ds(c * L, L)] for c in range(C)]
            zero16 = jnp.full((L,), 0, jnp.int32)
            zerov = jnp.zeros((L,), jnp.float32)
            iotap = iota * (NB * K)
            for j in range(NB):
                vch = [vrows[p][j, pl.ds(c * L, L)] for c in range(C)]
                for c in range(C):
                    str_s[p][j, pl.ds(c * L, L)] = zerov
                for g in range(K // L):
                    base = j * K + g * L
                    sv = sp_v[p][pl.ds(base, L)]
                    s0 = lax.gather(
                        sv, zero16[:, None], gdn, (1,),
                        mode=lax.GatherScatterMode.PROMISE_IN_BOUNDS)
                    u0 = tuple(rows[p][base, pl.ds(c * L, L)]
                               for c in range(C))

                    def edge(kk, carry):
                        s, u = carry[0], carry[1:]
                        kkv1 = zero16 + (kk + 1)
                        s_nxt = lax.gather(
                            sv, kkv1[:, None], gdn, (1,),
                            mode=lax.GatherScatterMode.PROMISE_IN_BOUNDS)
                        u_nxt = tuple(
                            rows[p][base + kk + 1, pl.ds(c * L, L)]
                            for c in range(C))
                        m = [jnp.maximum(vch[c] + s * u[c], 0.0)
                             for c in range(C)]
                        for c in range(C):
                            plsc.addupdate(str_s[p].at[j, pl.ds(c * L, L)],
                                           m[c])
                        p0 = m[0] * wch[0]
                        p1 = m[1] * wch[1]
                        for c in range(2, C, 2):
                            p0 = p0 + m[c] * wch[c]
                        for c in range(3, C, 2):
                            p1 = p1 + m[c] * wch[c]
                        plsc.store_scatter(
                            parts, [iota, (zero16 + kk) + base], p0 + p1)
                        return (s_nxt,) + u_nxt

                    lax.fori_loop(0, L, edge, (s0,) + u0)
                for g in range(K // L):
                    r0 = b_v[...]
                    r1 = parts[0, pl.ds(j * K + g * L, L)]
                    for l in range(1, L, 2):
                        r0 = r0 + parts[l, pl.ds(j * K + g * L, L)]
                    for l in range(2, L, 2):
                        r1 = r1 + parts[l, pl.ds(j * K + g * L, L)]
                    pos_s[p][j, pl.ds(g * L, L)] = r0 + r1

        start_aux(wid, 0)
        start_aux(wid + NW, 1)
        start_aux(wid + 2 * NW, 2)
        wait_aux(0)
        start_gather(0)
        wait_aux(1)
        start_gather(1)

        def trip_body(ip, carry):
            for par in (0, 1, 2):
                i = ip * 3 + par
                b = wid + i * NW
                nxt2 = (par + 2) % 3

                @pl.when(b < NUM_BATCHES)
                def _():
                    wait_gather(par)

                    @pl.when(b + 2 * NW < NUM_BATCHES)
                    def _():
                        wait_aux(nxt2)
                        start_gather(nxt2)

                    @pl.when(i >= 3)
                    def _():
                        wait_out(par)

                    compute(par)
                    start_out(b, par)

                    @pl.when(b + 3 * NW < NUM_BATCHES)
                    def _():
                        start_aux(b + 3 * NW, par)

            return carry

        lax.fori_loop(0, TRIPS, trip_body, 0)
        wait_out(0)
        wait_out(1)
        wait_out(2)

    return k(u_feat, v_feat, src, sp, wvec, bvec)


def kernel(feature, sp_dist, dists_max, edge_src, edge_dst, anchor_eid,
           W_u, b_u, W_v, b_v, W_out, b_out):
    u_feat, v_feat = _tc_linear(feature, W_u, b_u, W_v, b_v)
    src = edge_src.astype(jnp.int32).reshape(E // 128, 128)
    wvec = W_out.reshape(D) * jnp.float32(K)
    bvec = jnp.broadcast_to(b_out.astype(jnp.float32), (L,))
    out_position, out_structure = _sc_edges(
        u_feat, v_feat, src, sp_dist, wvec, bvec)
    return (out_position, out_structure)


# speedup vs baseline: 1.0074x; 1.0074x over previous
import functools

import jax
import jax.numpy as jnp
from jax import lax
from jax.experimental import pallas as pl
from jax.experimental.pallas import tpu as pltpu
from jax.experimental.pallas import tpu_sc as plsc

N = 10000
K = 32
D = 128
E = N * K
L = 16
C = D // L
NB = 8
IR = NB * K // 128
NUM_BATCHES = N // NB
NW = 32
BATCHES_PER_W = -(-NUM_BATCHES // NW)
BR = 1000


def _tc_linear(feature, W_u, b_u, W_v, b_v):
    def body(x_ref, wu_ref, bu_ref, wv_ref, bv_ref, u_ref, v_ref):
        x = x_ref[...]
        u_ref[...] = (lax.dot_general(
            x, wu_ref[...], (((1,), (1,)), ((), ())),
            preferred_element_type=jnp.float32) + bu_ref[...]) * (1.0 / K)
        v_ref[...] = (lax.dot_general(
            x, wv_ref[...], (((1,), (1,)), ((), ())),
            preferred_element_type=jnp.float32) + bv_ref[...]) * (1.0 / K)

    return pl.pallas_call(
        body,
        grid=(N // BR,),
        in_specs=[
            pl.BlockSpec((BR, D), lambda i: (i, 0)),
            pl.BlockSpec((D, D), lambda i: (0, 0)),
            pl.BlockSpec((1, D), lambda i: (0, 0)),
            pl.BlockSpec((D, D), lambda i: (0, 0)),
            pl.BlockSpec((1, D), lambda i: (0, 0)),
        ],
        out_specs=[
            pl.BlockSpec((BR, D), lambda i: (i, 0)),
            pl.BlockSpec((BR, D), lambda i: (i, 0)),
        ],
        out_shape=[
            jax.ShapeDtypeStruct((N, D), jnp.float32),
            jax.ShapeDtypeStruct((N, D), jnp.float32),
        ],
    )(feature, W_u, b_u.reshape(1, D), W_v, b_v.reshape(1, D))


def _sc_edges(u_feat, v_feat, src, sp, wvec, bvec):
    mesh = plsc.VectorSubcoreMesh(core_axis_name="c", subcore_axis_name="s")
    PAIRS = (BATCHES_PER_W + 1) // 2

    @functools.partial(
        pl.kernel,
        out_type=[
            jax.ShapeDtypeStruct((N, K), jnp.float32),
            jax.ShapeDtypeStruct((N, D), jnp.float32),
        ],
        mesh=mesh,
        compiler_params=pltpu.CompilerParams(needs_layout_passes=False),
        scratch_types=[
            pltpu.VMEM((IR, 128), jnp.int32),
            pltpu.VMEM((IR, 128), jnp.int32),
            pltpu.VMEM((NB * K,), jnp.float32),
            pltpu.VMEM((NB * K,), jnp.float32),
            pltpu.VMEM((NB, D), jnp.float32),
            pltpu.VMEM((NB, D), jnp.float32),
            pltpu.VMEM((NB * K + 1, D), jnp.float32),
            pltpu.VMEM((NB * K + 1, D), jnp.float32),
            pltpu.VMEM((L, NB * K), jnp.float32),
            pltpu.VMEM((D,), jnp.float32),
            pltpu.VMEM((L,), jnp.float32),
            pltpu.VMEM((NB, K), jnp.float32),
            pltpu.VMEM((NB, K), jnp.float32),
            pltpu.VMEM((NB, D), jnp.float32),
            pltpu.VMEM((NB, D), jnp.float32),
            pltpu.SemaphoreType.DMA,
            pltpu.SemaphoreType.DMA,
            pltpu.SemaphoreType.DMA,
            pltpu.SemaphoreType.DMA,
            pltpu.SemaphoreType.DMA,
            pltpu.SemaphoreType.DMA,
        ],
    )
    def k(u_hbm, v_hbm, src_hbm, sp_hbm, w_hbm, b_hbm, pos_hbm, str_hbm,
          idx_v0, idx_v1, sp_v0, sp_v1, vr0, vr1, rows0, rows1,
          parts, w_v, b_v, pos_s0, pos_s1, str_s0, str_s1,
          sem_a0, sem_a1, sem_g0, sem_g1, sem_o0, sem_o1):
        idx_v = (idx_v0, idx_v1)
        sp_v = (sp_v0, sp_v1)
        vrows = (vr0, vr1)
        rows = (rows0, rows1)
        pos_s = (pos_s0, pos_s1)
        str_s = (str_s0, str_s1)
        sem_a = (sem_a0, sem_a1)
        sem_g = (sem_g0, sem_g1)
        sem_o = (sem_o0, sem_o1)

        wid = lax.axis_index("s") * 2 + lax.axis_index("c")
        pltpu.sync_copy(w_hbm, w_v)
        pltpu.sync_copy(b_hbm, b_v)
        iota = lax.iota(jnp.int32, L)

        def start_aux(b, p):
            nb = b * NB
            pltpu.async_copy(src_hbm.at[pl.ds(b * IR, IR)], idx_v[p], sem_a[p])
            pltpu.async_copy(sp_hbm.at[pl.ds(nb * K, NB * K)], sp_v[p], sem_a[p])
            pltpu.async_copy(v_hbm.at[pl.ds(nb, NB)], vrows[p], sem_a[p])

        def wait_aux(p):
            pltpu.make_async_copy(src_hbm.at[pl.ds(0, IR)], idx_v[p], sem_a[p]).wait()
            pltpu.make_async_copy(sp_hbm.at[pl.ds(0, NB * K)], sp_v[p], sem_a[p]).wait()
            pltpu.make_async_copy(v_hbm.at[pl.ds(0, NB)], vrows[p], sem_a[p]).wait()

        def start_gather(p):
            for h in range(IR):
                pltpu.async_copy(u_hbm.at[idx_v[p].at[h]],
                                 rows[p].at[pl.ds(h * 128, 128)], sem_g[p])

        def wait_gather(p):
            for h in range(IR):
                pltpu.make_async_copy(u_hbm.at[idx_v[p].at[h]],
                                      rows[p].at[pl.ds(h * 128, 128)],
                                      sem_g[p]).wait()

        def start_out(b, p):
            nb = b * NB
            pltpu.async_copy(pos_s[p], pos_hbm.at[pl.ds(nb, NB)], sem_o[p])
            pltpu.async_copy(str_s[p], str_hbm.at[pl.ds(nb, NB)], sem_o[p])

        def wait_out(p):
            pltpu.make_async_copy(pos_s[p], pos_hbm.at[pl.ds(0, NB)], sem_o[p]).wait()
            pltpu.make_async_copy(str_s[p], str_hbm.at[pl.ds(0, NB)], sem_o[p]).wait()

        gdn = lax.GatherDimensionNumbers(
            offset_dims=(), collapsed_slice_dims=(0,), start_index_map=(0,))

        def compute(p):
            wch = [w_v[pl.ds(c * L, L)] for c in range(C)]
            zero16 = jnp.full((L,), 0, jnp.int32)
            zerov = jnp.zeros((L,), jnp.float32)
            iotap = iota * (NB * K)
            for j in range(NB):
                vch = [vrows[p][j, pl.ds(c * L, L)] for c in range(C)]
                for c in range(C):
                    str_s[p][j, pl.ds(c * L, L)] = zerov
                for g in range(K // L):
                    base = j * K + g * L
                    sv = sp_v[p][pl.ds(base, L)]
                    s0 = lax.gather(
                        sv, zero16[:, None], gdn, (1,),
                        mode=lax.GatherScatterMode.PROMISE_IN_BOUNDS)
                    u0 = tuple(rows[p][base, pl.ds(c * L, L)]
                               for c in range(C))

                    def edge(kk, carry):
                        s, u = carry[0], carry[1:]
                        kkv1 = zero16 + (kk + 1)
                        s_nxt = lax.gather(
                            sv, kkv1[:, None], gdn, (1,),
                            mode=lax.GatherScatterMode.PROMISE_IN_BOUNDS)
                        u_nxt = tuple(
                            rows[p][base + kk + 1, pl.ds(c * L, L)]
                            for c in range(C))
                        m = [jnp.maximum(vch[c] + s * u[c], 0.0)
                             for c in range(C)]
                        for c in range(C):
                            plsc.addupdate(str_s[p].at[j, pl.ds(c * L, L)],
                                           m[c])
                        p0 = m[0] * wch[0]
                        p1 = m[1] * wch[1]
                        for c in range(2, C, 2):
                            p0 = p0 + m[c] * wch[c]
                        for c in range(3, C, 2):
                            p1 = p1 + m[c] * wch[c]
                        plsc.store_scatter(
                            parts, [iota, (zero16 + kk) + base], p0 + p1)
                        return (s_nxt,) + u_nxt

                    lax.fori_loop(0, L, edge, (s0,) + u0)
                for g in range(K // L):
                    r0 = b_v[...]
                    r1 = parts[0, pl.ds(j * K + g * L, L)]
                    for l in range(1, L, 2):
                        r0 = r0 + parts[l, pl.ds(j * K + g * L, L)]
                    for l in range(2, L, 2):
                        r1 = r1 + parts[l, pl.ds(j * K + g * L, L)]
                    pos_s[p][j, pl.ds(g * L, L)] = r0 + r1

        start_aux(wid, 0)
        wait_aux(0)
        start_gather(0)
        start_aux(wid + NW, 1)

        def pair_body(ip, carry):
            for par in (0, 1):
                i = ip * 2 + par
                b = wid + i * NW

                @pl.when(b < NUM_BATCHES)
                def _():
                    wait_gather(par)

                    @pl.when(b + NW < NUM_BATCHES)
                    def _():
                        wait_aux(1 - par)
                        start_gather(1 - par)

                    @pl.when(ip >= 1)
                    def _():
                        wait_out(par)

                    compute(par)
                    start_out(b, par)

                    @pl.when(b + 2 * NW < NUM_BATCHES)
                    def _():
                        start_aux(b + 2 * NW, par)

            return carry

        lax.fori_loop(0, PAIRS, pair_body, 0)
        wait_out(0)
        wait_out(1)

    return k(u_feat, v_feat, src, sp, wvec, bvec)


def kernel(feature, sp_dist, dists_max, edge_src, edge_dst, anchor_eid,
           W_u, b_u, W_v, b_v, W_out, b_out):
    u_feat, v_feat = _tc_linear(feature, W_u, b_u, W_v, b_v)
    src = edge_src.astype(jnp.int32).reshape(E // 128, 128)
    wvec = W_out.reshape(D) * jnp.float32(K)
    bvec = jnp.broadcast_to(b_out.astype(jnp.float32), (L,))
    out_position, out_structure = _sc_edges(
        u_feat, v_feat, src, sp_dist, wvec, bvec)
    return (out_position, out_structure)
